# baseline (speedup 1.0000x reference)
import functools

import jax
import jax.numpy as jnp
from jax import lax
from jax.experimental import pallas as pl
from jax.experimental.pallas import tpu as pltpu
from jax.experimental.pallas import tpu_sc as plsc

N = 8192
D = 2048
E = 64
K = 8
CAP = float(int(1.25 * N / E))

EPAD = 128
TILE = 1024
NBLK = N // TILE
NEG = -1e30

NC = 2
NS = 16
NW = NC * NS
ROWS_PER_W = N // NW
CH = 32
NCHUNK = ROWS_PER_W // CH
LANES = 16


@functools.partial(
    pl.kernel,
    mesh=plsc.VectorSubcoreMesh(core_axis_name="c", subcore_axis_name="s"),
    out_type=jax.ShapeDtypeStruct((N, D), jnp.float32),
    scratch_types=[pltpu.VMEM((CH, D), jnp.float32)],
)
def _zeros_sc(out_hbm, zbuf):
    wid = lax.axis_index("s") * NC + lax.axis_index("c")

    def zrow(i, carry):
        r = i // (D // LANES)
        c = i % (D // LANES)
        zbuf[r, pl.ds(c * LANES, LANES)] = jnp.zeros((LANES,), jnp.float32)
        return carry

    lax.fori_loop(0, CH * (D // LANES), zrow, 0)

    base = wid * ROWS_PER_W

    def chunk(j, carry):
        pltpu.sync_copy(zbuf, out_hbm.at[pl.ds(base + j * CH, CH)])
        return carry

    lax.fori_loop(0, NCHUNK, chunk, 0)


def _router_body(x_ref, wrt_ref, br_ref, pcounts_ref, pconf_ref):
    logits = jnp.dot(x_ref[...], wrt_ref[...],
                     preferred_element_type=jnp.float32) + br_ref[...]

    m = jnp.max(logits, axis=1, keepdims=True)
    pconf_ref[...] = jnp.sum(m).reshape(1, 1, 1)
    vals = logits
    for _ in range(K - 1):
        vals = jnp.where(vals >= m, jnp.float32(NEG), vals)
        m = jnp.max(vals, axis=1, keepdims=True)
    member = (logits >= m).astype(jnp.float32)
    pcounts_ref[...] = jnp.sum(member, axis=0).reshape(1, 1, EPAD)


def _stats_body(pcounts_ref, pconf_ref, loss_ref, dist_ref, conf_ref):
    counts = jnp.sum(pcounts_ref[...], axis=(0, 1)).reshape(1, EPAD)
    load = jnp.minimum(counts, jnp.float32(CAP))
    s = jnp.sum(load)
    dist = load / (s + jnp.float32(1e-8))
    dist_ref[...] = dist
    loss_ref[...] = jnp.sum(dist * jnp.log(dist + jnp.float32(1e-8))).reshape(1, 1)
    conf_ref[...] = jnp.sum(pconf_ref[...]).reshape(1, 1) * jnp.float32(1.0 / N)


def kernel(x, Wr, br, W1, b1, W2, b2):
    del W1, b1, W2, b2
    wrt = jnp.pad(Wr.T, ((0, 0), (0, EPAD - E)))
    brp = jnp.pad(br.reshape(1, E), ((0, 0), (0, EPAD - E)),
                  constant_values=NEG)

    out = _zeros_sc()

    pcounts, pconf = pl.pallas_call(
        _router_body,
        grid=(NBLK,),
        in_specs=[
            pl.BlockSpec((TILE, D), lambda i: (i, 0)),
            pl.BlockSpec((D, EPAD), lambda i: (0, 0)),
            pl.BlockSpec((1, EPAD), lambda i: (0, 0)),
        ],
        out_specs=[
            pl.BlockSpec((1, 1, EPAD), lambda i: (i, 0, 0)),
            pl.BlockSpec((1, 1, 1), lambda i: (i, 0, 0)),
        ],
        out_shape=[
            jax.ShapeDtypeStruct((NBLK, 1, EPAD), jnp.float32),
            jax.ShapeDtypeStruct((NBLK, 1, 1), jnp.float32),
        ],
        compiler_params=pltpu.CompilerParams(
            dimension_semantics=("parallel",)),
    )(x, wrt, brp)

    loss, dist, conf = pl.pallas_call(
        _stats_body,
        out_shape=[
            jax.ShapeDtypeStruct((1, 1), jnp.float32),
            jax.ShapeDtypeStruct((1, EPAD), jnp.float32),
            jax.ShapeDtypeStruct((1, 1), jnp.float32),
        ],
    )(pcounts, pconf)

    return (out,
            loss.reshape(()),
            dist[0, :E],
            conf.reshape(()))


# speedup vs baseline: 1.0008x; 1.0008x over previous
import functools

import jax
import jax.numpy as jnp
from jax import lax
from jax.experimental import pallas as pl
from jax.experimental.pallas import tpu as pltpu
from jax.experimental.pallas import tpu_sc as plsc

N = 8192
D = 2048
E = 64
K = 8
CAP = float(int(1.25 * N / E))

EPAD = 128
TILE = 1024
NBLK = N // TILE
NEG = -1e30

NC = 2
NS = 16
NW = NC * NS
ROWS_PER_W = N // NW
CH = 32
NCHUNK = ROWS_PER_W // CH
LANES = 16


@functools.partial(
    pl.kernel,
    mesh=plsc.VectorSubcoreMesh(core_axis_name="c", subcore_axis_name="s"),
    out_type=jax.ShapeDtypeStruct((N, D), jnp.float32),
    scratch_types=[pltpu.VMEM((CH, D), jnp.float32),
                   pltpu.SemaphoreType.DMA],
)
def _zeros_sc(out_hbm, zbuf, sem):
    wid = lax.axis_index("s") * NC + lax.axis_index("c")

    def zrow(i, carry):
        r = i // (D // LANES)
        c = i % (D // LANES)
        zbuf[r, pl.ds(c * LANES, LANES)] = jnp.zeros((LANES,), jnp.float32)
        return carry

    lax.fori_loop(0, CH * (D // LANES), zrow, 0)

    base = wid * ROWS_PER_W

    for j in range(NCHUNK):
        pltpu.async_copy(zbuf, out_hbm.at[pl.ds(base + j * CH, CH)], sem)
    for j in range(NCHUNK):
        pltpu.make_async_copy(
            zbuf, out_hbm.at[pl.ds(base + j * CH, CH)], sem).wait()


def _router_body(x_ref, wrt_ref, br_ref, pcounts_ref, pconf_ref):
    logits = jnp.dot(x_ref[...], wrt_ref[...],
                     preferred_element_type=jnp.float32) + br_ref[...]

    m = jnp.max(logits, axis=1, keepdims=True)
    pconf_ref[...] = jnp.sum(m).reshape(1, 1, 1)
    vals = logits
    for _ in range(K - 1):
        vals = jnp.where(vals >= m, jnp.float32(NEG), vals)
        m = jnp.max(vals, axis=1, keepdims=True)
    member = (logits >= m).astype(jnp.float32)
    pcounts_ref[...] = jnp.sum(member, axis=0).reshape(1, 1, EPAD)


def _stats_body(pcounts_ref, pconf_ref, loss_ref, dist_ref, conf_ref):
    counts = jnp.sum(pcounts_ref[...], axis=(0, 1)).reshape(1, EPAD)
    load = jnp.minimum(counts, jnp.float32(CAP))
    s = jnp.sum(load)
    dist = load / (s + jnp.float32(1e-8))
    dist_ref[...] = dist
    loss_ref[...] = jnp.sum(dist * jnp.log(dist + jnp.float32(1e-8))).reshape(1, 1)
    conf_ref[...] = jnp.sum(pconf_ref[...]).reshape(1, 1) * jnp.float32(1.0 / N)


def kernel(x, Wr, br, W1, b1, W2, b2):
    del W1, b1, W2, b2
    wrt = jnp.pad(Wr.T, ((0, 0), (0, EPAD - E)))
    brp = jnp.pad(br.reshape(1, E), ((0, 0), (0, EPAD - E)),
                  constant_values=NEG)

    out = _zeros_sc()

    pcounts, pconf = pl.pallas_call(
        _router_body,
        grid=(NBLK,),
        in_specs=[
            pl.BlockSpec((TILE, D), lambda i: (i, 0)),
            pl.BlockSpec((D, EPAD), lambda i: (0, 0)),
            pl.BlockSpec((1, EPAD), lambda i: (0, 0)),
        ],
        out_specs=[
            pl.BlockSpec((1, 1, EPAD), lambda i: (i, 0, 0)),
            pl.BlockSpec((1, 1, 1), lambda i: (i, 0, 0)),
        ],
        out_shape=[
            jax.ShapeDtypeStruct((NBLK, 1, EPAD), jnp.float32),
            jax.ShapeDtypeStruct((NBLK, 1, 1), jnp.float32),
        ],
        compiler_params=pltpu.CompilerParams(
            dimension_semantics=("parallel",)),
    )(x, wrt, brp)

    loss, dist, conf = pl.pallas_call(
        _stats_body,
        out_shape=[
            jax.ShapeDtypeStruct((1, 1), jnp.float32),
            jax.ShapeDtypeStruct((1, EPAD), jnp.float32),
            jax.ShapeDtypeStruct((1, 1), jnp.float32),
        ],
    )(pcounts, pconf)

    return (out,
            loss.reshape(()),
            dist[0, :E],
            conf.reshape(()))


# speedup vs baseline: 1.0012x; 1.0004x over previous
import functools

import jax
import jax.numpy as jnp
from jax import lax
from jax.experimental import pallas as pl
from jax.experimental.pallas import tpu as pltpu
from jax.experimental.pallas import tpu_sc as plsc

N = 8192
D = 2048
E = 64
K = 8
CAP = float(int(1.25 * N / E))

EPAD = 128
TILE = 1024
NBLK = N // TILE
NEG = -1e30

NC = 2
NS = 16
NW = NC * NS
ROWS_PER_W = N // NW
CH = 32
NCHUNK = ROWS_PER_W // CH
LANES = 16


@functools.partial(
    pl.kernel,
    mesh=plsc.VectorSubcoreMesh(core_axis_name="c", subcore_axis_name="s"),
    out_type=jax.ShapeDtypeStruct((N, D), jnp.float32),
    scratch_types=[pltpu.VMEM((CH, D), jnp.float32),
                   pltpu.SemaphoreType.DMA],
)
def _zeros_sc(out_hbm, zbuf, sem):
    wid = lax.axis_index("s") * NC + lax.axis_index("c")

    def zrow(i, carry):
        r = i // (D // LANES)
        c = i % (D // LANES)
        zbuf[r, pl.ds(c * LANES, LANES)] = jnp.zeros((LANES,), jnp.float32)
        return carry

    lax.fori_loop(0, CH * (D // LANES), zrow, 0)

    base = wid * ROWS_PER_W

    for j in range(NCHUNK):
        pltpu.async_copy(zbuf, out_hbm.at[pl.ds(base + j * CH, CH)], sem)
    for j in range(NCHUNK):
        pltpu.make_async_copy(
            zbuf, out_hbm.at[pl.ds(base + j * CH, CH)], sem).wait()


def _router_body(x_ref, wrt_ref, br_ref, pcounts_ref, pconf_ref):
    logits = jnp.dot(x_ref[...], wrt_ref[...],
                     preferred_element_type=jnp.float32) + br_ref[...]

    m = jnp.max(logits, axis=1, keepdims=True)
    pconf_ref[...] = jnp.sum(m).reshape(1, 1, 1)
    vals = logits
    for _ in range(K - 1):
        vals = jnp.where(vals >= m, jnp.float32(NEG), vals)
        m = jnp.max(vals, axis=1, keepdims=True)
    member = (logits >= m).astype(jnp.float32)
    pcounts_ref[...] = jnp.sum(member, axis=0).reshape(1, 1, EPAD)


def _stats_body(pcounts_ref, pconf_ref, loss_ref, dist_ref, conf_ref):
    counts = jnp.sum(pcounts_ref[...], axis=(0, 1)).reshape(1, EPAD)
    load = jnp.minimum(counts, jnp.float32(CAP))
    s = jnp.sum(load)
    dist = load / (s + jnp.float32(1e-8))
    dist_ref[...] = dist
    loss_ref[...] = jnp.sum(dist * jnp.log(dist + jnp.float32(1e-8))).reshape(1, 1)
    conf_ref[...] = jnp.sum(pconf_ref[...]).reshape(1, 1) * jnp.float32(1.0 / N)


def kernel(x, Wr, br, W1, b1, W2, b2):
    del W1, b1, W2, b2
    wrt = jnp.pad(Wr.T, ((0, 0), (0, EPAD - E)))
    brp = jnp.pad(br.reshape(1, E), ((0, 0), (0, EPAD - E)),
                  constant_values=NEG)

    pcounts, pconf = pl.pallas_call(
        _router_body,
        grid=(NBLK,),
        in_specs=[
            pl.BlockSpec((TILE, D), lambda i: (i, 0)),
            pl.BlockSpec((D, EPAD), lambda i: (0, 0)),
            pl.BlockSpec((1, EPAD), lambda i: (0, 0)),
        ],
        out_specs=[
            pl.BlockSpec((1, 1, EPAD), lambda i: (i, 0, 0)),
            pl.BlockSpec((1, 1, 1), lambda i: (i, 0, 0)),
        ],
        out_shape=[
            jax.ShapeDtypeStruct((NBLK, 1, EPAD), jnp.float32),
            jax.ShapeDtypeStruct((NBLK, 1, 1), jnp.float32),
        ],
        compiler_params=pltpu.CompilerParams(
            dimension_semantics=("parallel",)),
    )(x, wrt, brp)

    out = _zeros_sc()

    loss, dist, conf = pl.pallas_call(
        _stats_body,
        out_shape=[
            jax.ShapeDtypeStruct((1, 1), jnp.float32),
            jax.ShapeDtypeStruct((1, EPAD), jnp.float32),
            jax.ShapeDtypeStruct((1, 1), jnp.float32),
        ],
    )(pcounts, pconf)

    return (out,
            loss.reshape(()),
            dist[0, :E],
            conf.reshape(()))


# speedup vs baseline: 1.2174x; 1.2160x over previous
import jax
import jax.numpy as jnp
from jax.experimental import pallas as pl
from jax.experimental.pallas import tpu as pltpu

N = 8192
D = 2048
E = 64
K = 8
CAP = float(int(1.25 * N / E))

EPAD = 128
TILE = 1024
NBLK = N // TILE
NEG = -1e30


def _body(x_ref, wrt_ref, br_ref,
          out_ref, loss_ref, dist_ref, conf_ref,
          counts_ref, csum_ref):
    i = pl.program_id(0)

    out_ref[...] = jnp.zeros_like(out_ref)

    logits = jnp.dot(x_ref[...], wrt_ref[...],
                     preferred_element_type=jnp.float32) + br_ref[...]

    m = jnp.max(logits, axis=1, keepdims=True)
    conf_tile = jnp.sum(m)
    vals = logits
    for _ in range(K - 1):
        vals = jnp.where(vals >= m, jnp.float32(NEG), vals)
        m = jnp.max(vals, axis=1, keepdims=True)
    member = (logits >= m).astype(jnp.float32)
    counts_tile = jnp.sum(member, axis=0, keepdims=True)

    @pl.when(i == 0)
    def _():
        counts_ref[...] = jnp.zeros_like(counts_ref)
        csum_ref[...] = jnp.zeros_like(csum_ref)

    counts_ref[...] += counts_tile
    csum_ref[...] += conf_tile

    @pl.when(i == NBLK - 1)
    def _():
        load = jnp.minimum(counts_ref[...], jnp.float32(CAP))
        s = jnp.sum(load)
        dist = load / (s + jnp.float32(1e-8))
        dist_ref[...] = dist
        loss_ref[...] = jnp.sum(dist * jnp.log(dist + jnp.float32(1e-8))).reshape(1, 1)
        conf_ref[...] = csum_ref[...] * jnp.float32(1.0 / N)


def kernel(x, Wr, br, W1, b1, W2, b2):
    del W1, b1, W2, b2
    wrt = jnp.pad(Wr.T, ((0, 0), (0, EPAD - E)))
    brp = jnp.pad(br.reshape(1, E), ((0, 0), (0, EPAD - E)),
                  constant_values=NEG)

    out, loss, dist, conf = pl.pallas_call(
        _body,
        grid=(NBLK,),
        in_specs=[
            pl.BlockSpec((TILE, D), lambda i: (i, 0)),
            pl.BlockSpec((D, EPAD), lambda i: (0, 0)),
            pl.BlockSpec((1, EPAD), lambda i: (0, 0)),
        ],
        out_specs=[
            pl.BlockSpec((TILE, D), lambda i: (i, 0)),
            pl.BlockSpec((1, 1), lambda i: (0, 0)),
            pl.BlockSpec((1, EPAD), lambda i: (0, 0)),
            pl.BlockSpec((1, 1), lambda i: (0, 0)),
        ],
        out_shape=[
            jax.ShapeDtypeStruct((N, D), jnp.float32),
            jax.ShapeDtypeStruct((1, 1), jnp.float32),
            jax.ShapeDtypeStruct((1, EPAD), jnp.float32),
            jax.ShapeDtypeStruct((1, 1), jnp.float32),
        ],
        scratch_shapes=[
            pltpu.VMEM((1, EPAD), jnp.float32),
            pltpu.VMEM((1, 1), jnp.float32),
        ],
    )(x, wrt, brp)

    return (out,
            loss.reshape(()),
            dist[0, :E],
            conf.reshape(()))
